# baseline (speedup 1.0000x reference)
import functools

import jax
import jax.numpy as jnp
from jax import lax
from jax.experimental import pallas as pl
from jax.experimental.pallas import tpu as pltpu
from jax.experimental.pallas import tpu_sc as plsc

BATCH = 4096
HIST = 50
EMBED_DIM = 64
N = BATCH * HIST

_info = plsc.get_sparse_core_info()
NUM_CORES = _info.num_cores
NUM_SUBCORES = _info.num_subcores
NW = NUM_CORES * NUM_SUBCORES

GB = 128
NG = N // NW // GB
G = 16
NGRP = GB // G

_mesh = plsc.VectorSubcoreMesh(core_axis_name="c", subcore_axis_name="s")


@functools.partial(
    pl.kernel,
    mesh=_mesh,
    out_type=jax.ShapeDtypeStruct((N, EMBED_DIM), jnp.float32),
    scratch_types=[
        pltpu.VMEM((NG, GB), jnp.int32),
        pltpu.VMEM((2, G, 8, EMBED_DIM), jnp.float32),
        pltpu.VMEM((2, GB, EMBED_DIM), jnp.float32),
        pltpu.SemaphoreType.DMA,
        pltpu.SemaphoreType.DMA,
        pltpu.SemaphoreType.DMA,
    ],
    compiler_params=pltpu.CompilerParams(use_tc_tiling_on_sc=True),
)
def _gather_kernel(idx_hbm, table_hbm, out_hbm, idx_v, tiles_v, stage_v,
                   sem_t0, sem_t1, sem_o):
    wid = lax.axis_index("s") * NUM_CORES + lax.axis_index("c")
    pltpu.sync_copy(idx_hbm.at[wid], idx_v)
    sems = (sem_t0, sem_t1)

    def fire(v, tb):
        for l in range(G):
            t = v[l] >> 3
            pltpu.async_copy(table_hbm.at[t], tiles_v.at[tb, l], sems[tb])

    def drain(tb):
        for l in range(G):
            pltpu.make_async_copy(table_hbm.at[0], tiles_v.at[tb, l],
                                  sems[tb]).wait()

    def extract(v, tb, sb, m0):
        for l in range(G):
            s = v[l] & 7
            for kk in range(EMBED_DIM // 16):
                stage_v[sb, m0 + l, pl.ds(kk * 16, 16)] = (
                    tiles_v[tb, l, s, pl.ds(kk * 16, 16)])

    def chunk(j, carry):
        sb = j % 2
        v0 = idx_v[j, pl.ds(0, G)]
        fire(v0, 0)

        def group2(gg, vcur):
            for tb in range(2):
                g = gg * 2 + tb
                vnext = idx_v[j, pl.ds((g + 1) % NGRP * G, G)]

                @pl.when(g < NGRP - 1)
                def _():
                    fire(vnext, (tb + 1) % 2)

                drain(tb)
                extract(vcur, tb, sb, g * G)
                vcur = vnext
            return vcur

        lax.fori_loop(0, NGRP // 2, group2, v0)

        dst = out_hbm.at[pl.ds((wid * NG + j) * GB, GB)]

        @pl.when(j >= 2)
        def _():
            prev = out_hbm.at[pl.ds((wid * NG + j - 2) * GB, GB)]
            pltpu.make_async_copy(stage_v.at[sb], prev, sem_o).wait()

        pltpu.async_copy(stage_v.at[sb], dst, sem_o)
        return carry

    lax.fori_loop(0, NG, chunk, 0)
    for j in (NG - 2, NG - 1):
        dst = out_hbm.at[pl.ds((wid * NG + j) * GB, GB)]
        pltpu.make_async_copy(stage_v.at[j % 2], dst, sem_o).wait()


def kernel(x, table):
    t3 = table.reshape(125000, 8, EMBED_DIM)
    idx = x.reshape(NW, NG, GB).astype(jnp.int32)
    out = _gather_kernel(idx, t3)
    return out.reshape(BATCH, HIST, EMBED_DIM)


# speedup vs baseline: 1.4270x; 1.4270x over previous
import functools

import jax
import jax.numpy as jnp
from jax import lax
from jax.experimental import pallas as pl
from jax.experimental.pallas import tpu as pltpu
from jax.experimental.pallas import tpu_sc as plsc

BATCH = 4096
HIST = 50
EMBED_DIM = 64
N = BATCH * HIST

_info = plsc.get_sparse_core_info()
NUM_CORES = _info.num_cores
NUM_SUBCORES = _info.num_subcores
NW = NUM_CORES * NUM_SUBCORES

GB = 128
NG = N // NW // GB
G = 16
NGRP = GB // G

_mesh = plsc.VectorSubcoreMesh(core_axis_name="c", subcore_axis_name="s")


@functools.partial(
    pl.kernel,
    mesh=_mesh,
    out_type=jax.ShapeDtypeStruct((N, EMBED_DIM), jnp.float32),
    scratch_types=[
        pltpu.VMEM((NG, GB), jnp.int32),
        pltpu.VMEM((2, GB, EMBED_DIM), jnp.float32),
        pltpu.SemaphoreType.DMA,
        pltpu.SemaphoreType.DMA,
        pltpu.SemaphoreType.DMA,
    ],
    compiler_params=pltpu.CompilerParams(use_tc_tiling_on_sc=True),
)
def _gather_kernel(idx_hbm, table_hbm, out_hbm, idx_v, stage_v,
                   sem_t0, sem_t1, sem_o):
    wid = lax.axis_index("s") * NUM_CORES + lax.axis_index("c")
    pltpu.sync_copy(idx_hbm.at[wid], idx_v)
    sems = (sem_t0, sem_t1)

    def fire(v, tb, sb, m0):
        for l in range(G):
            r = v[l]
            pltpu.async_copy(table_hbm.at[r >> 3, r & 7],
                             stage_v.at[sb, m0 + l], sems[tb])

    def drain(tb, sb, m0):
        for l in range(G):
            pltpu.make_async_copy(table_hbm.at[0, 0],
                                  stage_v.at[sb, m0 + l], sems[tb]).wait()

    def chunk(j, carry):
        sb = j % 2

        @pl.when(j >= 2)
        def _():
            prev = out_hbm.at[pl.ds((wid * NG + j - 2) * GB, GB)]
            pltpu.make_async_copy(stage_v.at[sb], prev, sem_o).wait()

        def group2(gg, c2):
            for tb in range(2):
                g = gg * 2 + tb
                v = idx_v[j, pl.ds(g * G, G)]
                fire(v, tb, sb, g * G)
            for tb in range(2):
                g = gg * 2 + tb
                drain(tb, sb, g * G)
            return c2

        lax.fori_loop(0, NGRP // 2, group2, 0)

        dst = out_hbm.at[pl.ds((wid * NG + j) * GB, GB)]
        pltpu.async_copy(stage_v.at[sb], dst, sem_o)
        return carry

    lax.fori_loop(0, NG, chunk, 0)
    for j in (NG - 2, NG - 1):
        dst = out_hbm.at[pl.ds((wid * NG + j) * GB, GB)]
        pltpu.make_async_copy(stage_v.at[j % 2], dst, sem_o).wait()


def kernel(x, table):
    t3 = table.reshape(125000, 8, EMBED_DIM)
    idx = x.reshape(NW, NG, GB).astype(jnp.int32)
    out = _gather_kernel(idx, t3)
    return out.reshape(BATCH, HIST, EMBED_DIM)


# speedup vs baseline: 1.4322x; 1.0037x over previous
import functools

import jax
import jax.numpy as jnp
from jax import lax
from jax.experimental import pallas as pl
from jax.experimental.pallas import tpu as pltpu
from jax.experimental.pallas import tpu_sc as plsc

BATCH = 4096
HIST = 50
EMBED_DIM = 64
N = BATCH * HIST

_info = plsc.get_sparse_core_info()
NUM_CORES = _info.num_cores
NUM_SUBCORES = _info.num_subcores
NW = NUM_CORES * NUM_SUBCORES

GB = 128
NG = N // NW // GB
G = 16
NGRP = GB // G

_mesh = plsc.VectorSubcoreMesh(core_axis_name="c", subcore_axis_name="s")


@functools.partial(
    pl.kernel,
    mesh=_mesh,
    out_type=jax.ShapeDtypeStruct((N, EMBED_DIM), jnp.float32),
    scratch_types=[
        pltpu.VMEM((NG, GB), jnp.int32),
        pltpu.VMEM((2, GB, EMBED_DIM), jnp.float32),
        pltpu.SemaphoreType.DMA,
        pltpu.SemaphoreType.DMA,
        pltpu.SemaphoreType.DMA,
    ],
    compiler_params=pltpu.CompilerParams(use_tc_tiling_on_sc=True),
)
def _gather_kernel(idx_hbm, table_hbm, out_hbm, idx_v, stage_v,
                   sem_t0, sem_t1, sem_o):
    wid = lax.axis_index("s") * NUM_CORES + lax.axis_index("c")
    pltpu.sync_copy(idx_hbm.at[wid], idx_v)
    sems = (sem_t0, sem_t1)

    def fire(v, tb, sb, m0):
        for l in range(G):
            r = v[l]
            pltpu.async_copy(table_hbm.at[r >> 3, r & 7],
                             stage_v.at[sb, m0 + l], sems[tb])

    def drain(tb, sb, m0):
        pltpu.make_async_copy(out_hbm.at[pl.ds(0, G)],
                              stage_v.at[sb, pl.ds(m0, G)], sems[tb]).wait()

    def chunk(j, carry):
        sb = j % 2

        @pl.when(j >= 2)
        def _():
            prev = out_hbm.at[pl.ds((wid * NG + j - 2) * GB, GB)]
            pltpu.make_async_copy(stage_v.at[sb], prev, sem_o).wait()

        def group2(gg, c2):
            for tb in range(2):
                g = gg * 2 + tb
                v = idx_v[j, pl.ds(g * G, G)]
                fire(v, tb, sb, g * G)
            for tb in range(2):
                g = gg * 2 + tb
                drain(tb, sb, g * G)
            return c2

        lax.fori_loop(0, NGRP // 2, group2, 0)

        dst = out_hbm.at[pl.ds((wid * NG + j) * GB, GB)]
        pltpu.async_copy(stage_v.at[sb], dst, sem_o)
        return carry

    lax.fori_loop(0, NG, chunk, 0)
    for j in (NG - 2, NG - 1):
        dst = out_hbm.at[pl.ds((wid * NG + j) * GB, GB)]
        pltpu.make_async_copy(stage_v.at[j % 2], dst, sem_o).wait()


def kernel(x, table):
    t3 = table.reshape(125000, 8, EMBED_DIM)
    idx = x.reshape(NW, NG, GB).astype(jnp.int32)
    out = _gather_kernel(idx, t3)
    return out.reshape(BATCH, HIST, EMBED_DIM)


# speedup vs baseline: 1.6740x; 1.1688x over previous
import functools

import jax
import jax.numpy as jnp
from jax import lax
from jax.experimental import pallas as pl
from jax.experimental.pallas import tpu as pltpu
from jax.experimental.pallas import tpu_sc as plsc

BATCH = 4096
HIST = 50
EMBED_DIM = 64
N = BATCH * HIST

_info = plsc.get_sparse_core_info()
NUM_CORES = _info.num_cores
NUM_SUBCORES = _info.num_subcores
NW = NUM_CORES * NUM_SUBCORES

KB = 8
CL = KB * HIST
NC = BATCH // NW // KB
G = 16
NGRP = CL // G

_mesh = plsc.VectorSubcoreMesh(core_axis_name="c", subcore_axis_name="s")


@functools.partial(
    pl.kernel,
    mesh=_mesh,
    out_type=jax.ShapeDtypeStruct((BATCH, HIST, EMBED_DIM), jnp.float32),
    scratch_types=[
        pltpu.VMEM((NC, CL), jnp.int32),
        pltpu.VMEM((2, KB, HIST, EMBED_DIM), jnp.float32),
        pltpu.SemaphoreType.DMA,
        pltpu.SemaphoreType.DMA,
        pltpu.SemaphoreType.DMA,
    ],
    compiler_params=pltpu.CompilerParams(use_tc_tiling_on_sc=True),
)
def _gather_kernel(idx_hbm, table_hbm, out_hbm, idx_v, stage_v,
                   sem_t0, sem_t1, sem_o):
    wid = lax.axis_index("s") * NUM_CORES + lax.axis_index("c")
    pltpu.sync_copy(idx_hbm.at[wid], idx_v)
    sems = (sem_t0, sem_t1)

    def fire(j, g, tb, sb):
        v = idx_v[j, pl.ds(g * G, G)]
        for l in range(G):
            i = g * G + l
            bb = (i * 5243) >> 18
            hh = i - bb * 50
            r = v[l]
            pltpu.async_copy(table_hbm.at[r >> 3, r & 7],
                             stage_v.at[sb, bb, hh], sems[tb])

    def drain(tb, sb):
        pltpu.make_async_copy(out_hbm.at[0, pl.ds(0, 16), :],
                              stage_v.at[sb, 0, pl.ds(0, 16)],
                              sems[tb]).wait()

    def chunk(j, carry):
        sb = j % 2

        @pl.when(j >= 2)
        def _():
            b0 = (wid * NC + j - 2) * KB
            prev = out_hbm.at[pl.ds(b0, KB)]
            pltpu.make_async_copy(stage_v.at[sb], prev, sem_o).wait()

        fire(j, 0, 0, sb)

        def pair(p, c2):
            for tb01 in range(2):
                g = p * 2 + tb01
                fire(j, g + 1, (tb01 + 1) % 2, sb)
                drain(tb01, sb)
            return c2

        lax.fori_loop(0, (NGRP - 1) // 2, pair, 0)
        drain(0, sb)

        b0 = (wid * NC + j) * KB
        pltpu.async_copy(stage_v.at[sb], out_hbm.at[pl.ds(b0, KB)], sem_o)
        return carry

    lax.fori_loop(0, NC, chunk, 0)
    for j in (NC - 2, NC - 1):
        b0 = (wid * NC + j) * KB
        pltpu.make_async_copy(stage_v.at[j % 2],
                              out_hbm.at[pl.ds(b0, KB)], sem_o).wait()


def kernel(x, table):
    t3 = table.reshape(125000, 8, EMBED_DIM)
    idx = x.reshape(NW, NC, CL).astype(jnp.int32)
    return _gather_kernel(idx, t3)
